# baseline (speedup 1.0000x reference)
import functools

import jax
import jax.numpy as jnp
from jax import lax
from jax.experimental import pallas as pl
from jax.experimental.pallas import tpu as pltpu
from jax.experimental.pallas import tpu_sc as plsc

VOCAB = 1000000
EMBED_DIM = 64
OUTPUT_DIM = 2
BATCH = 4096
SEQ = 200

_PW = 16
_T_BLK = 4096
_NB = (VOCAB + _T_BLK - 1) // _T_BLK
_VP = _NB * _T_BLK

_INFO = plsc.get_sparse_core_info()
_NC = _INFO.num_cores
_NS = _INFO.num_subcores
_NW = _NC * _NS
_B_PER_W = BATCH // _NW
_K = 8


def _tc_tablew(tableT, Wp):
    def body(t_ref, w_ref, o_ref):
        blk = lax.dot_general(
            t_ref[...], w_ref[...], (((0,), (0,)), ((), ())),
            preferred_element_type=jnp.float32,
            precision=lax.Precision.HIGHEST,
        )
        rpb = _T_BLK * _PW // 128
        blk3 = jnp.reshape(blk, (rpb, 8, _PW))
        o_ref[...] = jnp.concatenate([blk3[:, j, :] for j in range(8)], axis=-1)

    rows_per_blk = _T_BLK * _PW // 128
    return pl.pallas_call(
        body,
        grid=(_NB,),
        in_specs=[
            pl.BlockSpec((EMBED_DIM, _T_BLK), lambda i: (0, i)),
            pl.BlockSpec((EMBED_DIM, _PW), lambda i: (0, 0)),
        ],
        out_specs=pl.BlockSpec((rows_per_blk, 128), lambda i: (i, 0)),
        out_shape=jax.ShapeDtypeStruct((_NB * rows_per_blk, 128), jnp.float32),
    )(tableT, Wp)


def _sc_pooled(ids_t, tablew):
    mesh = plsc.VectorSubcoreMesh(core_axis_name="c", subcore_axis_name="s")

    @functools.partial(
        pl.kernel,
        mesh=mesh,
        out_type=jax.ShapeDtypeStruct((BATCH, _PW), jnp.float32),
        scratch_types=[
            pltpu.VMEM((SEQ, _B_PER_W), jnp.int32),
            pltpu.VMEM((_B_PER_W, _PW), jnp.float32),
            pltpu.SemaphoreType.DMA,
        ],
        compiler_params=pltpu.CompilerParams(use_tc_tiling_on_sc=False),
    )
    def k(ids_hbm, tab_hbm, out_hbm, idx_v, acc_v, sem):
        wid = lax.axis_index("s") * _NC + lax.axis_index("c")
        pltpu.sync_copy(ids_hbm.at[:, pl.ds(wid * _B_PER_W, _B_PER_W)], idx_v)

        zero = jnp.zeros((16,), jnp.float32)

        def zero_body(i, carry):
            acc_v[i, pl.ds(0, 16)] = zero
            return carry

        lax.fori_loop(0, _B_PER_W, zero_body, 0)

        def fire(r):
            return pltpu.async_copy(
                tab_hbm.at[idx_v.at[r]], acc_v, sem, add=True
            )

        for j in range(_K):
            fire(j)

        def chunk_body(i, carry):
            for j in range(_K):
                fire(i * _K + j)
            for j in range(_K):
                pltpu.make_async_copy(tab_hbm.at[idx_v.at[0]], acc_v, sem).wait()
            return carry

        lax.fori_loop(1, SEQ // _K, chunk_body, 0)
        for j in range(_K):
            pltpu.make_async_copy(tab_hbm.at[idx_v.at[0]], acc_v, sem).wait()

        pltpu.sync_copy(acc_v, out_hbm.at[pl.ds(wid * _B_PER_W, _B_PER_W)])

    return k(ids_t, tablew)


def _tc_out(pooled, b):
    def body(s_ref, b_ref, o_ref):
        o_ref[...] = s_ref[:, 0:OUTPUT_DIM] + b_ref[...]

    return pl.pallas_call(
        body,
        out_shape=jax.ShapeDtypeStruct((BATCH, OUTPUT_DIM), jnp.float32),
    )(pooled, b.reshape(1, OUTPUT_DIM))


@jax.jit
def kernel(ids, table, W, b):
    tableT = jnp.transpose(table)
    Wp = jnp.pad(W * (1.0 / SEQ), ((0, 0), (0, _PW - OUTPUT_DIM)))
    tablew = _tc_tablew(tableT, Wp)
    ids_t = jnp.transpose(ids.astype(jnp.int32))
    pooled = _sc_pooled(ids_t, jnp.reshape(tablew, (_VP, _PW)))
    return _tc_out(pooled, b)


# speedup vs baseline: 2.0061x; 2.0061x over previous
import functools

import jax
import jax.numpy as jnp
from jax import lax
from jax.experimental import pallas as pl
from jax.experimental.pallas import tpu as pltpu
from jax.experimental.pallas import tpu_sc as plsc

VOCAB = 1000000
EMBED_DIM = 64
OUTPUT_DIM = 2
BATCH = 4096
SEQ = 200

_PW = 32
_T_BLK = 4096
_NJ = 128 // _PW
_SUB = _T_BLK // _NJ
_SHIFT = _SUB.bit_length() - 1
_NB = (VOCAB + _T_BLK - 1) // _T_BLK

_INFO = plsc.get_sparse_core_info()
_NC = _INFO.num_cores
_NS = _INFO.num_subcores
_NW = _NC * _NS
_B_PER_W = BATCH // _NW
_K = 8


def _tc_tablew(tableT, Wp):
    def body(t_ref, w_ref, o_ref):
        acc = None
        for j in range(_NJ):
            part = lax.dot_general(
                t_ref[:, j * _SUB:(j + 1) * _SUB],
                w_ref[:, j * 128:(j + 1) * 128],
                (((0,), (0,)), ((), ())),
                preferred_element_type=jnp.float32,
            )
            acc = part if acc is None else acc + part
        o_ref[...] = acc

    return pl.pallas_call(
        body,
        grid=(_NB,),
        in_specs=[
            pl.BlockSpec((EMBED_DIM, _T_BLK), lambda i: (0, i)),
            pl.BlockSpec((EMBED_DIM, _NJ * 128), lambda i: (0, 0)),
        ],
        out_specs=pl.BlockSpec((_SUB, _NJ * _PW), lambda i: (i, 0)),
        out_shape=jax.ShapeDtypeStruct((_NB * _SUB, _NJ * _PW), jnp.float32),
    )(tableT, Wp)


def _sc_pooled(ids_t, tablew):
    mesh = plsc.VectorSubcoreMesh(core_axis_name="c", subcore_axis_name="s")

    @functools.partial(
        pl.kernel,
        mesh=mesh,
        out_type=jax.ShapeDtypeStruct((BATCH, _PW), jnp.float32),
        scratch_types=[
            pltpu.VMEM((SEQ, _B_PER_W), jnp.int32),
            pltpu.VMEM((_B_PER_W, _PW), jnp.float32),
            pltpu.SemaphoreType.DMA,
        ],
        compiler_params=pltpu.CompilerParams(use_tc_tiling_on_sc=False),
    )
    def k(ids_hbm, tab_hbm, out_hbm, idx_v, acc_v, sem):
        wid = lax.axis_index("s") * _NC + lax.axis_index("c")
        pltpu.sync_copy(ids_hbm.at[:, pl.ds(wid * _B_PER_W, _B_PER_W)], idx_v)

        def remap_body(r, carry):
            for g in range(_B_PER_W // 16):
                t = idx_v[r, pl.ds(g * 16, 16)]
                q = (
                    (t & ~(_T_BLK - 1))
                    + (t & (_SUB - 1)) * _NJ
                    + ((t >> _SHIFT) & (_NJ - 1))
                )
                idx_v[r, pl.ds(g * 16, 16)] = q
            return carry

        lax.fori_loop(0, SEQ, remap_body, 0)

        zero = jnp.zeros((16,), jnp.float32)

        def zero_body(i, carry):
            for d in range(_PW // 16):
                acc_v[i, pl.ds(d * 16, 16)] = zero
            return carry

        lax.fori_loop(0, _B_PER_W, zero_body, 0)

        def fire(r):
            return pltpu.async_copy(
                tab_hbm.at[idx_v.at[r]], acc_v, sem, add=True
            )

        for j in range(_K):
            fire(j)

        def chunk_body(i, carry):
            for j in range(_K):
                fire(i * _K + j)
            for j in range(_K):
                pltpu.make_async_copy(tab_hbm.at[idx_v.at[0]], acc_v, sem).wait()
            return carry

        lax.fori_loop(1, SEQ // _K, chunk_body, 0)
        for j in range(_K):
            pltpu.make_async_copy(tab_hbm.at[idx_v.at[0]], acc_v, sem).wait()

        pltpu.sync_copy(acc_v, out_hbm.at[pl.ds(wid * _B_PER_W, _B_PER_W)])

    return k(ids_t, tablew)


def _tc_out(pooled, b):
    def body(s_ref, b_ref, o_ref):
        o_ref[...] = s_ref[:, 0:OUTPUT_DIM] + b_ref[...]

    return pl.pallas_call(
        body,
        out_shape=jax.ShapeDtypeStruct((BATCH, OUTPUT_DIM), jnp.float32),
    )(pooled, b.reshape(1, OUTPUT_DIM))


@jax.jit
def kernel(ids, table, W, b):
    tableT = jnp.transpose(table)
    Ws = W * (1.0 / SEQ)
    Wbig = jnp.zeros((_NJ, EMBED_DIM, 128), jnp.float32)
    for j in range(_NJ):
        Wbig = Wbig.at[j, :, j * _PW:j * _PW + OUTPUT_DIM].set(Ws)
    Wbig = jnp.reshape(jnp.transpose(Wbig, (1, 0, 2)), (EMBED_DIM, _NJ * 128))
    tablew = _tc_tablew(tableT, Wbig)
    ids_t = jnp.transpose(ids.astype(jnp.int32))
    pooled = _sc_pooled(
        ids_t, jnp.reshape(tablew, (_NB * _SUB * _NJ, _PW))
    )
    return _tc_out(pooled, b)


# speedup vs baseline: 2.9032x; 1.4472x over previous
import functools

import jax
import jax.numpy as jnp
from jax import lax
from jax.experimental import pallas as pl
from jax.experimental.pallas import tpu as pltpu
from jax.experimental.pallas import tpu_sc as plsc

VOCAB = 1000000
EMBED_DIM = 64
OUTPUT_DIM = 2
BATCH = 4096
SEQ = 200

_PW = 32
_T_BLK = 16384
_NJ = 128 // _PW
_SUB = _T_BLK // _NJ
_SHIFT = _SUB.bit_length() - 1
_NB = (VOCAB + _T_BLK - 1) // _T_BLK

_INFO = plsc.get_sparse_core_info()
_NC = _INFO.num_cores
_NS = _INFO.num_subcores
_NW = _NC * _NS
_B_PER_W = BATCH // _NW
_K = 8


def _tc_tablew(tableT, Wp):
    def body(t_ref, w_ref, o_ref):
        acc = None
        for j in range(_NJ):
            part = lax.dot_general(
                t_ref[:, j * _SUB:(j + 1) * _SUB],
                w_ref[:, j * 128:(j + 1) * 128],
                (((0,), (0,)), ((), ())),
                preferred_element_type=jnp.float32,
            )
            acc = part if acc is None else acc + part
        o_ref[...] = acc

    return pl.pallas_call(
        body,
        grid=(_NB,),
        in_specs=[
            pl.BlockSpec((EMBED_DIM, _T_BLK), lambda i: (0, i)),
            pl.BlockSpec((EMBED_DIM, _NJ * 128), lambda i: (0, 0)),
        ],
        out_specs=pl.BlockSpec((_SUB, _NJ * _PW), lambda i: (i, 0)),
        out_shape=jax.ShapeDtypeStruct((_NB * _SUB, _NJ * _PW), jnp.float32),
    )(tableT, Wp)


def _sc_pooled(ids_t, tablew):
    mesh = plsc.VectorSubcoreMesh(core_axis_name="c", subcore_axis_name="s")

    @functools.partial(
        pl.kernel,
        mesh=mesh,
        out_type=jax.ShapeDtypeStruct((BATCH, _PW), jnp.float32),
        scratch_types=[
            pltpu.VMEM((SEQ, _B_PER_W), jnp.int32),
            pltpu.VMEM((_B_PER_W, _PW), jnp.float32),
            pltpu.SemaphoreType.DMA,
        ],
        compiler_params=pltpu.CompilerParams(use_tc_tiling_on_sc=False),
    )
    def k(ids_hbm, tab_hbm, out_hbm, idx_v, acc_v, sem):
        wid = lax.axis_index("s") * _NC + lax.axis_index("c")
        pltpu.sync_copy(ids_hbm.at[:, pl.ds(wid * _B_PER_W, _B_PER_W)], idx_v)

        def remap_body(r, carry):
            for g in range(_B_PER_W // 16):
                t = idx_v[r, pl.ds(g * 16, 16)]
                q = (
                    (t & ~(_T_BLK - 1))
                    + (t & (_SUB - 1)) * _NJ
                    + ((t >> _SHIFT) & (_NJ - 1))
                )
                idx_v[r, pl.ds(g * 16, 16)] = q
            return carry

        lax.fori_loop(0, SEQ, remap_body, 0)

        zero = jnp.zeros((16,), jnp.float32)

        def zero_body(i, carry):
            for d in range(_PW // 16):
                acc_v[i, pl.ds(d * 16, 16)] = zero
            return carry

        lax.fori_loop(0, _B_PER_W, zero_body, 0)

        def fire(r):
            return pltpu.async_copy(
                tab_hbm.at[idx_v.at[r]], acc_v, sem, add=True
            )

        for j in range(_K):
            fire(j)

        def chunk_body(i, carry):
            for j in range(_K):
                fire(i * _K + j)
            for j in range(_K):
                pltpu.make_async_copy(tab_hbm.at[idx_v.at[0]], acc_v, sem).wait()
            return carry

        lax.fori_loop(1, SEQ // _K, chunk_body, 0)
        for j in range(_K):
            pltpu.make_async_copy(tab_hbm.at[idx_v.at[0]], acc_v, sem).wait()

        pltpu.sync_copy(acc_v, out_hbm.at[pl.ds(wid * _B_PER_W, _B_PER_W)])

    return k(ids_t, tablew)


def _tc_out(pooled, b):
    def body(s_ref, b_ref, o_ref):
        o_ref[...] = s_ref[:, 0:OUTPUT_DIM] + b_ref[...]

    return pl.pallas_call(
        body,
        out_shape=jax.ShapeDtypeStruct((BATCH, OUTPUT_DIM), jnp.float32),
    )(pooled, b.reshape(1, OUTPUT_DIM))


@jax.jit
def kernel(ids, table, W, b):
    tableT = jnp.transpose(table)
    Ws = W * (1.0 / SEQ)
    Wbig = jnp.zeros((_NJ, EMBED_DIM, 128), jnp.float32)
    for j in range(_NJ):
        Wbig = Wbig.at[j, :, j * _PW:j * _PW + OUTPUT_DIM].set(Ws)
    Wbig = jnp.reshape(jnp.transpose(Wbig, (1, 0, 2)), (EMBED_DIM, _NJ * 128))
    tablew = _tc_tablew(tableT, Wbig)
    ids_t = jnp.transpose(ids.astype(jnp.int32))
    pooled = _sc_pooled(
        ids_t, jnp.reshape(tablew, (_NB * _SUB * _NJ, _PW))
    )
    return _tc_out(pooled, b)


# speedup vs baseline: 3.1511x; 1.0854x over previous
import functools

import jax
import jax.numpy as jnp
from jax import lax
from jax.experimental import pallas as pl
from jax.experimental.pallas import tpu as pltpu
from jax.experimental.pallas import tpu_sc as plsc

VOCAB = 1000000
EMBED_DIM = 64
OUTPUT_DIM = 2
BATCH = 4096
SEQ = 200

_PW = 32
_T_BLK = 32768
_NJ = 128 // _PW
_SUB = _T_BLK // _NJ
_SHIFT = _SUB.bit_length() - 1
_NB = (VOCAB + _T_BLK - 1) // _T_BLK

_INFO = plsc.get_sparse_core_info()
_NC = _INFO.num_cores
_NS = _INFO.num_subcores
_NW = _NC * _NS
_B_PER_W = BATCH // _NW
_K = 8


def _tc_tablew(tableT, Wp):
    def body(t_ref, w_ref, o_ref):
        acc = None
        for j in range(_NJ):
            part = lax.dot_general(
                t_ref[:, j * _SUB:(j + 1) * _SUB],
                w_ref[:, j * 128:(j + 1) * 128],
                (((0,), (0,)), ((), ())),
                preferred_element_type=jnp.float32,
            )
            acc = part if acc is None else acc + part
        o_ref[...] = acc

    return pl.pallas_call(
        body,
        grid=(_NB,),
        in_specs=[
            pl.BlockSpec((EMBED_DIM, _T_BLK), lambda i: (0, i)),
            pl.BlockSpec((EMBED_DIM, _NJ * 128), lambda i: (0, 0)),
        ],
        out_specs=pl.BlockSpec((_SUB, _NJ * _PW), lambda i: (i, 0)),
        out_shape=jax.ShapeDtypeStruct((_NB * _SUB, _NJ * _PW), jnp.float32),
    )(tableT, Wp)


def _sc_pooled(ids_t, tablew):
    mesh = plsc.VectorSubcoreMesh(core_axis_name="c", subcore_axis_name="s")

    @functools.partial(
        pl.kernel,
        mesh=mesh,
        out_type=jax.ShapeDtypeStruct((BATCH, _PW), jnp.float32),
        scratch_types=[
            pltpu.VMEM((SEQ, _B_PER_W), jnp.int32),
            pltpu.VMEM((_B_PER_W, _PW), jnp.float32),
            pltpu.SemaphoreType.DMA,
        ],
        compiler_params=pltpu.CompilerParams(use_tc_tiling_on_sc=False),
    )
    def k(ids_hbm, tab_hbm, out_hbm, idx_v, acc_v, sem):
        wid = lax.axis_index("s") * _NC + lax.axis_index("c")
        pltpu.sync_copy(ids_hbm.at[:, pl.ds(wid * _B_PER_W, _B_PER_W)], idx_v)

        def remap_body(r, carry):
            for g in range(_B_PER_W // 16):
                t = idx_v[r, pl.ds(g * 16, 16)]
                q = (
                    (t & ~(_T_BLK - 1))
                    + (t & (_SUB - 1)) * _NJ
                    + ((t >> _SHIFT) & (_NJ - 1))
                )
                idx_v[r, pl.ds(g * 16, 16)] = q
            return carry

        lax.fori_loop(0, SEQ, remap_body, 0)

        zero = jnp.zeros((16,), jnp.float32)

        def zero_body(i, carry):
            for d in range(_PW // 16):
                acc_v[i, pl.ds(d * 16, 16)] = zero
            return carry

        lax.fori_loop(0, _B_PER_W, zero_body, 0)

        def fire(r):
            return pltpu.async_copy(
                tab_hbm.at[idx_v.at[r]], acc_v, sem, add=True
            )

        for j in range(_K):
            fire(j)

        def chunk_body(i, carry):
            for j in range(_K):
                fire(i * _K + j)
            for j in range(_K):
                pltpu.make_async_copy(tab_hbm.at[idx_v.at[0]], acc_v, sem).wait()
            return carry

        lax.fori_loop(1, SEQ // _K, chunk_body, 0)
        for j in range(_K):
            pltpu.make_async_copy(tab_hbm.at[idx_v.at[0]], acc_v, sem).wait()

        pltpu.sync_copy(acc_v, out_hbm.at[pl.ds(wid * _B_PER_W, _B_PER_W)])

    return k(ids_t, tablew)


def _tc_out(pooled, b):
    def body(s_ref, b_ref, o_ref):
        o_ref[...] = s_ref[:, 0:OUTPUT_DIM] + b_ref[...]

    return pl.pallas_call(
        body,
        out_shape=jax.ShapeDtypeStruct((BATCH, OUTPUT_DIM), jnp.float32),
    )(pooled, b.reshape(1, OUTPUT_DIM))


@jax.jit
def kernel(ids, table, W, b):
    tableT = jnp.transpose(table)
    Ws = W * (1.0 / SEQ)
    Wbig = jnp.zeros((_NJ, EMBED_DIM, 128), jnp.float32)
    for j in range(_NJ):
        Wbig = Wbig.at[j, :, j * _PW:j * _PW + OUTPUT_DIM].set(Ws)
    Wbig = jnp.reshape(jnp.transpose(Wbig, (1, 0, 2)), (EMBED_DIM, _NJ * 128))
    tablew = _tc_tablew(tableT, Wbig)
    ids_t = jnp.transpose(ids.astype(jnp.int32))
    pooled = _sc_pooled(
        ids_t, jnp.reshape(tablew, (_NB * _SUB * _NJ, _PW))
    )
    return _tc_out(pooled, b)


# speedup vs baseline: 3.2162x; 1.0207x over previous
import functools

import jax
import jax.numpy as jnp
from jax import lax
from jax.experimental import pallas as pl
from jax.experimental.pallas import tpu as pltpu
from jax.experimental.pallas import tpu_sc as plsc

VOCAB = 1000000
EMBED_DIM = 64
OUTPUT_DIM = 2
BATCH = 4096
SEQ = 200

_PW = 16
_T_BLK = 32768
_NJ = 128 // _PW
_SUB = _T_BLK // _NJ
_SHIFT = _SUB.bit_length() - 1
_NB = (VOCAB + _T_BLK - 1) // _T_BLK

_INFO = plsc.get_sparse_core_info()
_NC = _INFO.num_cores
_NS = _INFO.num_subcores
_NW = _NC * _NS
_B_PER_W = BATCH // _NW
_K = 8


def _tc_tablew(tableT, Wp):
    def body(t_ref, w_ref, o_ref):
        acc = None
        for j in range(_NJ):
            part = lax.dot_general(
                t_ref[:, j * _SUB:(j + 1) * _SUB],
                w_ref[:, j * 128:(j + 1) * 128],
                (((0,), (0,)), ((), ())),
                preferred_element_type=jnp.float32,
            )
            acc = part if acc is None else acc + part
        o_ref[...] = acc

    return pl.pallas_call(
        body,
        grid=(_NB,),
        in_specs=[
            pl.BlockSpec((EMBED_DIM, _T_BLK), lambda i: (0, i)),
            pl.BlockSpec((EMBED_DIM, _NJ * 128), lambda i: (0, 0)),
        ],
        out_specs=pl.BlockSpec((_SUB, _NJ * _PW), lambda i: (i, 0)),
        out_shape=jax.ShapeDtypeStruct((_NB * _SUB, _NJ * _PW), jnp.float32),
    )(tableT, Wp)


def _sc_pooled(ids_t, tablew):
    mesh = plsc.VectorSubcoreMesh(core_axis_name="c", subcore_axis_name="s")

    @functools.partial(
        pl.kernel,
        mesh=mesh,
        out_type=jax.ShapeDtypeStruct((BATCH, _PW), jnp.float32),
        scratch_types=[
            pltpu.VMEM((SEQ, _B_PER_W), jnp.int32),
            pltpu.VMEM((_B_PER_W, _PW), jnp.float32),
            pltpu.SemaphoreType.DMA,
        ],
        compiler_params=pltpu.CompilerParams(use_tc_tiling_on_sc=False),
    )
    def k(ids_hbm, tab_hbm, out_hbm, idx_v, acc_v, sem):
        wid = lax.axis_index("s") * _NC + lax.axis_index("c")
        pltpu.sync_copy(ids_hbm.at[:, pl.ds(wid * _B_PER_W, _B_PER_W)], idx_v)

        def remap_body(r, carry):
            for g in range(_B_PER_W // 16):
                t = idx_v[r, pl.ds(g * 16, 16)]
                q = (
                    (t & ~(_T_BLK - 1))
                    + (t & (_SUB - 1)) * _NJ
                    + ((t >> _SHIFT) & (_NJ - 1))
                )
                idx_v[r, pl.ds(g * 16, 16)] = q
            return carry

        lax.fori_loop(0, SEQ, remap_body, 0)

        zero = jnp.zeros((16,), jnp.float32)

        def zero_body(i, carry):
            for d in range(_PW // 16):
                acc_v[i, pl.ds(d * 16, 16)] = zero
            return carry

        lax.fori_loop(0, _B_PER_W, zero_body, 0)

        def fire(r):
            return pltpu.async_copy(
                tab_hbm.at[idx_v.at[r]], acc_v, sem, add=True
            )

        for j in range(_K):
            fire(j)

        def chunk_body(i, carry):
            for j in range(_K):
                fire(i * _K + j)
            for j in range(_K):
                pltpu.make_async_copy(tab_hbm.at[idx_v.at[0]], acc_v, sem).wait()
            return carry

        lax.fori_loop(1, SEQ // _K, chunk_body, 0)
        for j in range(_K):
            pltpu.make_async_copy(tab_hbm.at[idx_v.at[0]], acc_v, sem).wait()

        pltpu.sync_copy(acc_v, out_hbm.at[pl.ds(wid * _B_PER_W, _B_PER_W)])

    return k(ids_t, tablew)


def _tc_out(pooled, b):
    def body(s_ref, b_ref, o_ref):
        o_ref[...] = s_ref[:, 0:OUTPUT_DIM] + b_ref[...]

    return pl.pallas_call(
        body,
        out_shape=jax.ShapeDtypeStruct((BATCH, OUTPUT_DIM), jnp.float32),
    )(pooled, b.reshape(1, OUTPUT_DIM))


@jax.jit
def kernel(ids, table, W, b):
    tableT = jnp.transpose(table)
    Ws = W * (1.0 / SEQ)
    Wbig = jnp.zeros((_NJ, EMBED_DIM, 128), jnp.float32)
    for j in range(_NJ):
        Wbig = Wbig.at[j, :, j * _PW:j * _PW + OUTPUT_DIM].set(Ws)
    Wbig = jnp.reshape(jnp.transpose(Wbig, (1, 0, 2)), (EMBED_DIM, _NJ * 128))
    tablew = _tc_tablew(tableT, Wbig)
    ids_t = jnp.transpose(ids.astype(jnp.int32))
    pooled = _sc_pooled(
        ids_t, jnp.reshape(tablew, (_NB * _SUB * _NJ, _PW))
    )
    return _tc_out(pooled, b)
